# baseline (speedup 1.0000x reference)
import functools

import jax
import jax.numpy as jnp
from jax import lax
from jax.experimental import pallas as pl
from jax.experimental.pallas import tpu as pltpu, tpu_sc as plsc

_B = 16384
_D = 64
_NC = 2
_NS = 16
_NW = _NC * _NS
_BPW = _B // _NW
_CHUNK = 128
_NCHUNK = _BPW // _CHUNK


def _make_lookup():
  mesh = plsc.VectorSubcoreMesh(core_axis_name="c", subcore_axis_name="s")

  @functools.partial(
      pl.kernel,
      out_type=jax.ShapeDtypeStruct((_B, _D), jnp.float32),
      mesh=mesh,
      scratch_types=[
          pltpu.VMEM((_BPW,), jnp.int32),
          pltpu.VMEM((_BPW, _D), jnp.float32),
          pltpu.SemaphoreType.DMA,
      ],
      compiler_params=pltpu.CompilerParams(use_tc_tiling_on_sc=False),
  )
  def lookup(labels_hbm, table_hbm, out_hbm, idx_v, rows_v, sem):
    wid = lax.axis_index("s") * _NC + lax.axis_index("c")
    base = wid * _BPW
    pltpu.sync_copy(labels_hbm.at[pl.ds(base, _BPW)], idx_v)
    copies = []
    for j in range(_NCHUNK):
      copies.append(
          pltpu.async_copy(
              table_hbm.at[idx_v.at[pl.ds(j * _CHUNK, _CHUNK)]],
              rows_v.at[pl.ds(j * _CHUNK, _CHUNK)],
              sem,
          )
      )
    for c in copies:
      c.wait()
    pltpu.sync_copy(rows_v, out_hbm.at[pl.ds(base, _BPW)])

  return lookup


_lookup = _make_lookup()


@jax.jit
def kernel(labels, embedding_table):
  return _lookup(labels.astype(jnp.int32), embedding_table)


# speedup vs baseline: 3.9414x; 3.9414x over previous
import functools

import jax
import jax.numpy as jnp
from jax import lax
from jax.experimental import pallas as pl
from jax.experimental.pallas import tpu as pltpu, tpu_sc as plsc

_B = 16384
_D = 64
_V = 1000001
_NC = 2
_NS = 16
_NW = _NC * _NS
_NTC = (_V + 127) // 128
_RANGE = 245
_WTC = 4
_LANES = _WTC * 128
_NWIN = 62
_NOUTER = _NWIN // 2
_OUTR = _B + _NW


def _make_scan():
  mesh = plsc.VectorSubcoreMesh(core_axis_name="c", subcore_axis_name="s")

  @functools.partial(
      pl.kernel,
      out_type=jax.ShapeDtypeStruct((_OUTR, 128), jnp.float32),
      mesh=mesh,
      scratch_types=[
          pltpu.VMEM((_B + 16,), jnp.int32),
          pltpu.VMEM((_B + 16,), jnp.int32),
          pltpu.VMEM((_B + 16,), jnp.int32),
          pltpu.VMEM((2, _D, _LANES), jnp.float32),
          pltpu.VMEM((2, 16, 128), jnp.float32),
          pltpu.VMEM((2, 16), jnp.int32),
          pltpu.SemaphoreType.DMA,
          pltpu.SemaphoreType.DMA,
          pltpu.SemaphoreType.DMA,
      ],
      compiler_params=pltpu.CompilerParams(needs_layout_passes=False),
  )
  def scan(labels_hbm, tt_hbm, out_hbm, lwj, ml, mb, win, stage, pend,
           sem0, sem1, semo):
    wid = lax.axis_index("s") * _NC + lax.axis_index("c")
    iota = lax.iota(jnp.int32, 16)
    lo_tc = wid * _RANGE
    hi_tc = jnp.minimum(lo_tc + _RANGE, _NTC)

    def lane_off(k):
      c0 = jnp.minimum(lo_tc + k * _WTC, _NTC - _WTC)
      return c0 * 128

    def fetch(k, buf, sem):
      pltpu.async_copy(tt_hbm.at[:, pl.ds(lane_off(k), _LANES)], buf, sem)

    fetch(0, win.at[0], sem0)
    fetch(1, win.at[1], sem1)

    pltpu.sync_copy(labels_hbm, lwj.at[pl.ds(0, _B)])

    def pre(c, cnt):
      lv = lwj[pl.ds(c * 16, 16)]
      tc = lax.shift_right_logical(lv, 7)
      m = (tc >= lo_tc) & (tc < hi_tc)
      mi = jnp.where(m, 1, 0)
      pos = cnt + plsc.cumsum(mi) - 1
      plsc.store_scatter(ml, [pos], lv, mask=m)
      bv = c * 16 + iota
      plsc.store_scatter(mb, [pos], bv, mask=m)
      return cnt + jnp.sum(mi)

    cnt = lax.fori_loop(0, _B // 16, pre, 0)
    nch = (cnt + 15) // 16

    def window_pass(k, buf, sem, p):
      pltpu.make_async_copy(
          tt_hbm.at[:, pl.ds(0, _LANES)], buf, sem).wait()
      c0n = lo_tc + k * _WTC
      base_lane = c0n * 128

      def wfil(j, wcnt):
        jv = j * 16 + iota
        lv = ml[pl.ds(j * 16, 16)]
        tc = lax.shift_right_logical(lv, 7)
        m = (jv < cnt) & (tc >= c0n) & (tc < c0n + _WTC)
        mi = jnp.where(m, 1, 0)
        pos = wcnt + plsc.cumsum(mi) - 1
        plsc.store_scatter(lwj, [pos], jv, mask=m)
        return wcnt + jnp.sum(mi)

      wcnt = lax.fori_loop(0, nch, wfil, 0)

      def ext(i, p):
        j_s = plsc.load_gather(lwj, [jnp.broadcast_to(i, (16,))])
        l_s = plsc.load_gather(ml, [j_s])
        b_s = plsc.load_gather(mb, [j_s])
        lane = l_s - base_lane
        f = (p // 16) % 2
        slot = p % 16
        for ch in range(_D // 16):
          d_idx = ch * 16 + iota
          vals = plsc.load_gather(buf, [d_idx, lane])
          stage[f, slot, pl.ds(ch * 16, 16)] = vals
        plsc.store_scatter(
            pend.at[f], [jnp.broadcast_to(slot, (16,))], b_s, mask=iota == 0)
        p1 = p + 1

        @pl.when(p1 % 16 == 0)
        def _flush():
          pltpu.async_copy(stage.at[f], out_hbm.at[pend.at[f]], semo).wait()

        return p1

      p = lax.fori_loop(0, wcnt, ext, p)
      nk = k + 2

      @pl.when(nk < _NWIN)
      def _refetch():
        fetch(nk, buf, sem)

      return p

    def outer(t, p):
      p = window_pass(2 * t, win.at[0], sem0, p)
      p = window_pass(2 * t + 1, win.at[1], sem1, p)
      return p

    p = lax.fori_loop(0, _NOUTER, outer, 0)

    f = (p // 16) % 2
    rem = p % 16
    trash = jnp.broadcast_to(_B + wid, (16,))
    plsc.store_scatter(pend.at[f], [iota], trash, mask=iota >= rem)
    pltpu.async_copy(stage.at[f], out_hbm.at[pend.at[f]], semo).wait()

  return scan


_scan = _make_scan()


@jax.jit
def kernel(labels, embedding_table):
  out_wide = _scan(labels.astype(jnp.int32), embedding_table.T)
  return out_wide[:_B, :_D]
